# baseline (speedup 1.0000x reference)
import functools

import jax
import jax.numpy as jnp
from jax import lax
from jax.experimental import pallas as pl
from jax.experimental.pallas import tpu as pltpu
from jax.experimental.pallas import tpu_sc as plsc

NC = 2
NS = 16
NW = NC * NS
LANES = 16
CH = 128
NUM_GRAPHS = 128


def _seg_rows(table, row4, col4, zeros):
  n, d = table.shape
  npad = zeros.shape[0]
  _, nblocks, bpb, ch = row4.shape
  nb = 5
  rpb = bpb // nb
  nbodies = nblocks // 2
  rpt = npad // NS
  mesh = plsc.VectorSubcoreMesh(core_axis_name="c", subcore_axis_name="s")

  @functools.partial(
      pl.kernel, mesh=mesh,
      out_type=jax.ShapeDtypeStruct((NC, npad, d), jnp.float32),
      scratch_types=[
          pltpu.VMEM((bpb, ch), jnp.int32),
          pltpu.VMEM((bpb, ch), jnp.int32),
          pltpu.VMEM((bpb, ch), jnp.int32),
          pltpu.VMEM((bpb, ch), jnp.int32),
      ] + [pltpu.VMEM((ch, d), jnp.float32) for _ in range(nb)] + [
          pltpu.VMEM_SHARED((npad, d), jnp.float32),
      ] + [pltpu.SemaphoreType.DMA for _ in range(2 * nb + 2)])
  def k(table_h, row_h, col_h, zeros_h, out_h, rowA, colA, rowB, colB, *rest):
    bufs = rest[:nb]
    acc = rest[nb]
    gsems = rest[nb + 1:2 * nb + 1]
    ssems = rest[2 * nb + 1:3 * nb + 1]
    isemA, isemB = rest[3 * nb + 1:]
    c = lax.axis_index("c")
    s = lax.axis_index("s")
    wid = s * NC + c
    last = nblocks - 1
    pltpu.async_copy(row_h.at[wid, 0], rowA, isemA)
    pltpu.async_copy(col_h.at[wid, 0], colA, isemA)
    pltpu.async_copy(row_h.at[wid, 1], rowB, isemB)
    pltpu.async_copy(col_h.at[wid, 1], colB, isemB)
    pltpu.sync_copy(zeros_h.at[pl.ds(s * rpt, rpt)], acc.at[pl.ds(s * rpt, rpt)])
    plsc.subcore_barrier()

    def do_block(rowi, coli):
      gs = {}
      ss = {}
      for b in range(nb):
        gs[b] = pltpu.async_copy(table_h.at[rowi.at[b]], bufs[b], gsems[b])
      drained = set()
      for i in range(bpb):
        b = i % nb
        gs[b].wait()
        ss[i] = pltpu.async_copy(bufs[b], acc.at[coli.at[i]], ssems[b],
                                 add=True)
        j = i - nb + 1
        if j >= 0 and j + nb < bpb:
          ss[j].wait()
          drained.add(j)
          gs[j % nb] = pltpu.async_copy(table_h.at[rowi.at[j + nb]],
                                        bufs[j % nb], gsems[j % nb])
      for i in range(bpb):
        if i not in drained:
          ss[i].wait()

    def drain_fetch(rowi, coli, isem, nxt):
      pltpu.make_async_copy(row_h.at[wid, 0], rowi, isem).wait()
      pltpu.make_async_copy(col_h.at[wid, 0], coli, isem).wait()
      return nxt

    def body(j, _):
      drain_fetch(rowA, colA, isemA, None)
      do_block(rowA, colA)
      nxtA = jnp.minimum(2 * j + 2, last)
      pltpu.async_copy(row_h.at[wid, nxtA], rowA, isemA)
      pltpu.async_copy(col_h.at[wid, nxtA], colA, isemA)
      drain_fetch(rowB, colB, isemB, None)
      do_block(rowB, colB)
      nxtB = jnp.minimum(2 * j + 3, last)
      pltpu.async_copy(row_h.at[wid, nxtB], rowB, isemB)
      pltpu.async_copy(col_h.at[wid, nxtB], colB, isemB)
      return ()

    lax.fori_loop(0, nbodies, body, ())
    drain_fetch(rowA, colA, isemA, None)
    drain_fetch(rowB, colB, isemB, None)
    plsc.subcore_barrier()
    pltpu.sync_copy(acc.at[pl.ds(s * rpt, rpt)],
                    out_h.at[c, pl.ds(s * rpt, rpt)])

  return k(table, row4, col4, zeros)


def _seg_scalar(attn, row3, col3, zeros1):
  n = attn.shape[0]
  npad = zeros1.shape[0]
  nchunks = row3.shape[1]
  ch = row3.shape[2]
  nb = 25
  rounds = nchunks // nb
  rpt = npad // NS
  mesh = plsc.VectorSubcoreMesh(core_axis_name="c", subcore_axis_name="s")

  @functools.partial(
      pl.kernel, mesh=mesh,
      out_type=jax.ShapeDtypeStruct((NC * npad,), jnp.float32),
      scratch_types=[
          pltpu.VMEM((nchunks, ch), jnp.int32),
          pltpu.VMEM((nchunks, ch), jnp.int32),
          pltpu.VMEM((rpt,), jnp.float32),
      ] + [pltpu.VMEM((ch,), jnp.float32) for _ in range(nb)] + [
          pltpu.VMEM_SHARED((npad,), jnp.float32),
          pltpu.SemaphoreType.DMA,
          pltpu.SemaphoreType.DMA,
      ])
  def k(attn_h, row_h, col_h, zeros_h, out_h, rowi, coli, zbuf, *rest):
    bufs = rest[:nb]
    acc = rest[nb]
    gsem = rest[nb + 1]
    ssem = rest[nb + 2]
    c = lax.axis_index("c")
    s = lax.axis_index("s")
    wid = s * NC + c
    pltpu.sync_copy(row_h.at[wid], rowi)
    pltpu.sync_copy(col_h.at[wid], coli)
    pltpu.sync_copy(zeros_h.at[pl.ds(s * rpt, rpt)], zbuf)
    pltpu.sync_copy(zbuf, acc.at[pl.ds(s * rpt, rpt)])
    plsc.subcore_barrier()

    def body(j, _):
      base = j * nb
      gs = [pltpu.async_copy(attn_h.at[rowi.at[base + b]], bufs[b], gsem)
            for b in range(nb)]
      for b in range(nb):
        gs[b].wait()
      ss = [pltpu.async_copy(bufs[b], acc.at[coli.at[base + b]], ssem,
                             add=True) for b in range(nb)]
      for b in range(nb):
        ss[b].wait()
      return ()

    lax.fori_loop(0, rounds, body, ())
    plsc.subcore_barrier()
    pltpu.sync_copy(acc.at[pl.ds(s * rpt, rpt)], zbuf)
    pltpu.sync_copy(zbuf, out_h.at[pl.ds(c * npad + s * rpt, rpt)])

  return k(attn, row3, col3, zeros1).reshape(NC, npad)[:, :n]


def _bn_relu(t, g, b, n_rows):
  m = jnp.sum(t, axis=0, keepdims=True) * (1.0 / n_rows)
  v = jnp.sum((t - m) ** 2, axis=0, keepdims=True) * (1.0 / n_rows)
  return jnp.maximum((t - m) * lax.rsqrt(v + 1e-5) * g + b, 0.0)


def _dense_conv(x, p, wa, ba, g, be, wb, bb, watt, batt):
  n, d = x.shape
  h_dim = wb.shape[1]

  def body(x_r, p_r, wa_r, ba_r, g_r, be_r, wb_r, bb_r, watt_r, batt_r,
           h_out, a_out):
    xin = x_r[...] + p_r[0] + p_r[1]
    t = jnp.dot(xin, wa_r[...], preferred_element_type=jnp.float32) + ba_r[...]
    t = _bn_relu(t, g_r[...], be_r[...], float(n))
    h = jnp.dot(t, wb_r[...], preferred_element_type=jnp.float32) + bb_r[...]
    hr = jnp.maximum(h, 0.0)
    h_out[...] = hr
    logit = jnp.dot(hr, watt_r[...], preferred_element_type=jnp.float32)
    a_out[...] = jax.nn.sigmoid(logit + batt_r[...])

  return pl.pallas_call(
      body,
      out_shape=(jax.ShapeDtypeStruct((n, h_dim), jnp.float32),
                 jax.ShapeDtypeStruct((n, 1), jnp.float32)),
  )(x, p, wa, ba, g, be, wb, bb, watt, batt)


def _scale_rows(h, a, sp):
  n, d = h.shape

  def body(h_r, a_r, sp_r, out_r):
    s = jnp.sum(sp_r[...], axis=0)[:, None]
    out_r[...] = h_r[...] * (a_r[...] * s)

  return pl.pallas_call(
      body, out_shape=jax.ShapeDtypeStruct((n, d), jnp.float32))(h, a, sp)


def _final(h, a, sp, batch, wl1, bl1, gl, bel, wl2, bl2):
  n, d = h.shape
  o = wl2.shape[1]

  def body(h_r, a_r, sp_r, b_r, wl1_r, bl1_r, gl_r, bel_r, wl2_r, bl2_r,
           out_r):
    s = jnp.sum(sp_r[...], axis=0)[:, None]
    hp = h_r[...] * (a_r[...] * s)
    gids = lax.broadcasted_iota(jnp.int32, (1, NUM_GRAPHS), 1)
    onehot = (b_r[...] == gids).astype(jnp.float32)
    gph = lax.dot_general(onehot, hp, (((0,), (0,)), ((), ())),
                          preferred_element_type=jnp.float32)
    t = jnp.dot(gph, wl1_r[...], preferred_element_type=jnp.float32) + bl1_r[...]
    t = _bn_relu(t, gl_r[...], bel_r[...], float(NUM_GRAPHS))
    out = jnp.dot(t, wl2_r[...], preferred_element_type=jnp.float32) + bl2_r[...]
    mx = jnp.max(out, axis=1, keepdims=True)
    z = out - mx
    out_r[...] = z - jnp.log(jnp.sum(jnp.exp(z), axis=1, keepdims=True))

  return pl.pallas_call(
      body, out_shape=jax.ShapeDtypeStruct((NUM_GRAPHS, o), jnp.float32))(
          h, a, sp, batch, wl1, bl1, gl, bel, wl2, bl2)


def kernel(x, edge_index, batch, W1a, b1a, g1, be1, W1b, b1b, Wa1, ba1,
           W2a, b2a, g2, be2, W2b, b2b, Wa2, ba2, Wl1, bl1, gl, bel, Wl2, bl2):
  n, d = x.shape
  e = edge_index.shape[1]
  row = edge_index[0].astype(jnp.int32)
  col = edge_index[1].astype(jnp.int32)
  row4 = row.reshape(NW, 10, 25, 40)
  col4 = col.reshape(NW, 10, 25, 40)
  row3 = row.reshape(NW, 125, 80)
  col3 = col.reshape(NW, 125, 80)
  batch2 = batch.astype(jnp.int32).reshape(n, 1)
  npad = ((n + 127) // 128) * 128
  zeros = jnp.zeros((npad, d), jnp.float32)
  zeros1 = jnp.zeros((npad,), jnp.float32)

  p1 = _seg_rows(x, row4, col4, zeros)[:, :n]
  h1, a1 = _dense_conv(x, p1, W1a, b1a, g1, be1, W1b, b1b, Wa1, ba1)
  s1p = _seg_scalar(a1.reshape(n), row3, col3, zeros1)
  h1p = _scale_rows(h1, a1, s1p)
  p2 = _seg_rows(h1p, row4, col4, zeros)[:, :n]
  h2, a2 = _dense_conv(h1p, p2, W2a, b2a, g2, be2, W2b, b2b, Wa2, ba2)
  s2p = _seg_scalar(a2.reshape(n), row3, col3, zeros1)
  return _final(h2, a2, s2p, batch2, Wl1, bl1, gl, bel, Wl2, bl2)


# speedup vs baseline: 1.2492x; 1.2492x over previous
import functools

import jax
import jax.numpy as jnp
from jax import lax
from jax.experimental import pallas as pl
from jax.experimental.pallas import tpu as pltpu
from jax.experimental.pallas import tpu_sc as plsc

NC = 2
NS = 16
NW = NC * NS
LANES = 16
CH = 128
NUM_GRAPHS = 128


def _seg_rows(table, row4, col4, zeros):
  n, d = table.shape
  npad = zeros.shape[0]
  _, nblocks, bpb, ch = row4.shape
  nb = 5
  rpb = bpb // nb
  nbodies = nblocks // 2
  rpt = npad // NS
  mesh = plsc.VectorSubcoreMesh(core_axis_name="c", subcore_axis_name="s")

  @functools.partial(
      pl.kernel, mesh=mesh,
      out_type=jax.ShapeDtypeStruct((NC, npad, d), jnp.float32),
      scratch_types=[
          pltpu.VMEM((bpb, ch), jnp.int32),
          pltpu.VMEM((bpb, ch), jnp.int32),
          pltpu.VMEM((bpb, ch), jnp.int32),
          pltpu.VMEM((bpb, ch), jnp.int32),
      ] + [pltpu.VMEM((ch, d), jnp.float32) for _ in range(nb)] + [
          pltpu.VMEM_SHARED((npad, d), jnp.float32),
      ] + [pltpu.SemaphoreType.DMA for _ in range(2 * nb + 2)])
  def k(table_h, row_h, col_h, zeros_h, out_h, rowA, colA, rowB, colB, *rest):
    bufs = rest[:nb]
    acc = rest[nb]
    gsems = rest[nb + 1:2 * nb + 1]
    ssems = rest[2 * nb + 1:3 * nb + 1]
    isemA, isemB = rest[3 * nb + 1:]
    c = lax.axis_index("c")
    s = lax.axis_index("s")
    wid = s * NC + c
    last = nblocks - 1
    pltpu.async_copy(row_h.at[wid, 0], rowA, isemA)
    pltpu.async_copy(col_h.at[wid, 0], colA, isemA)
    pltpu.async_copy(row_h.at[wid, 1], rowB, isemB)
    pltpu.async_copy(col_h.at[wid, 1], colB, isemB)
    pltpu.sync_copy(zeros_h.at[pl.ds(s * rpt, rpt)], acc.at[pl.ds(s * rpt, rpt)])
    plsc.subcore_barrier()

    def do_block(rowi, coli):
      for r in range(rpb):
        gs = [pltpu.async_copy(table_h.at[rowi.at[r * nb + b]], bufs[b],
                               gsems[b]) for b in range(nb)]
        ss = []
        for b in range(nb):
          gs[b].wait()
          ss.append(pltpu.async_copy(bufs[b], acc.at[coli.at[r * nb + b]],
                                     ssems[b], add=True))
        for b in range(nb):
          ss[b].wait()

    def drain_fetch(rowi, coli, isem, nxt):
      pltpu.make_async_copy(row_h.at[wid, 0], rowi, isem).wait()
      pltpu.make_async_copy(col_h.at[wid, 0], coli, isem).wait()
      return nxt

    def body(j, _):
      drain_fetch(rowA, colA, isemA, None)
      do_block(rowA, colA)
      nxtA = jnp.minimum(2 * j + 2, last)
      pltpu.async_copy(row_h.at[wid, nxtA], rowA, isemA)
      pltpu.async_copy(col_h.at[wid, nxtA], colA, isemA)
      drain_fetch(rowB, colB, isemB, None)
      do_block(rowB, colB)
      nxtB = jnp.minimum(2 * j + 3, last)
      pltpu.async_copy(row_h.at[wid, nxtB], rowB, isemB)
      pltpu.async_copy(col_h.at[wid, nxtB], colB, isemB)
      return ()

    lax.fori_loop(0, nbodies, body, ())
    drain_fetch(rowA, colA, isemA, None)
    drain_fetch(rowB, colB, isemB, None)
    plsc.subcore_barrier()
    pltpu.sync_copy(acc.at[pl.ds(s * rpt, rpt)],
                    out_h.at[c, pl.ds(s * rpt, rpt)])

  return k(table, row4, col4, zeros)


def _seg_scalar(attn, row3, col3, zeros1):
  n = attn.shape[0]
  npad = zeros1.shape[0]
  nchunks = row3.shape[1]
  ch = row3.shape[2]
  nb = nchunks
  rpt = npad // NS
  mesh = plsc.VectorSubcoreMesh(core_axis_name="c", subcore_axis_name="s")

  @functools.partial(
      pl.kernel, mesh=mesh,
      out_type=jax.ShapeDtypeStruct((NC * npad,), jnp.float32),
      scratch_types=[
          pltpu.VMEM((nchunks, ch), jnp.int32),
          pltpu.VMEM((nchunks, ch), jnp.int32),
          pltpu.VMEM((rpt,), jnp.float32),
      ] + [pltpu.VMEM((ch,), jnp.float32) for _ in range(nb)] + [
          pltpu.VMEM_SHARED((npad,), jnp.float32),
          pltpu.SemaphoreType.DMA,
          pltpu.SemaphoreType.DMA,
      ])
  def k(attn_h, row_h, col_h, zeros_h, out_h, rowi, coli, zbuf, *rest):
    bufs = rest[:nb]
    acc = rest[nb]
    gsem = rest[nb + 1]
    ssem = rest[nb + 2]
    c = lax.axis_index("c")
    s = lax.axis_index("s")
    wid = s * NC + c
    pltpu.sync_copy(row_h.at[wid], rowi)
    pltpu.sync_copy(col_h.at[wid], coli)
    pltpu.sync_copy(zeros_h.at[pl.ds(s * rpt, rpt)], zbuf)
    pltpu.sync_copy(zbuf, acc.at[pl.ds(s * rpt, rpt)])
    plsc.subcore_barrier()

    gs = [pltpu.async_copy(attn_h.at[rowi.at[b]], bufs[b], gsem)
          for b in range(nb)]
    for b in range(nb):
      gs[b].wait()
    ss = [pltpu.async_copy(bufs[b], acc.at[coli.at[b]], ssem, add=True)
          for b in range(nb)]
    for b in range(nb):
      ss[b].wait()
    plsc.subcore_barrier()
    pltpu.sync_copy(acc.at[pl.ds(s * rpt, rpt)], zbuf)
    pltpu.sync_copy(zbuf, out_h.at[pl.ds(c * npad + s * rpt, rpt)])

  return k(attn, row3, col3, zeros1).reshape(NC, npad)[:, :n]


def _bn_relu(t, g, b, n_rows):
  m = jnp.sum(t, axis=0, keepdims=True) * (1.0 / n_rows)
  v = jnp.sum((t - m) ** 2, axis=0, keepdims=True) * (1.0 / n_rows)
  return jnp.maximum((t - m) * lax.rsqrt(v + 1e-5) * g + b, 0.0)


def _dense_conv(x, p, wa, ba, g, be, wb, bb, watt, batt):
  n, d = x.shape
  h_dim = wb.shape[1]

  def body(x_r, p_r, wa_r, ba_r, g_r, be_r, wb_r, bb_r, watt_r, batt_r,
           h_out, a_out):
    xin = x_r[...] + p_r[0] + p_r[1]
    t = jnp.dot(xin, wa_r[...], preferred_element_type=jnp.float32) + ba_r[...]
    t = _bn_relu(t, g_r[...], be_r[...], float(n))
    h = jnp.dot(t, wb_r[...], preferred_element_type=jnp.float32) + bb_r[...]
    hr = jnp.maximum(h, 0.0)
    h_out[...] = hr
    logit = jnp.dot(hr, watt_r[...], preferred_element_type=jnp.float32)
    a_out[...] = jax.nn.sigmoid(logit + batt_r[...])

  return pl.pallas_call(
      body,
      out_shape=(jax.ShapeDtypeStruct((n, h_dim), jnp.float32),
                 jax.ShapeDtypeStruct((n, 1), jnp.float32)),
  )(x, p, wa, ba, g, be, wb, bb, watt, batt)


def _scale_rows(h, a, sp):
  n, d = h.shape

  def body(h_r, a_r, sp_r, out_r):
    s = jnp.sum(sp_r[...], axis=0)[:, None]
    out_r[...] = h_r[...] * (a_r[...] * s)

  return pl.pallas_call(
      body, out_shape=jax.ShapeDtypeStruct((n, d), jnp.float32))(h, a, sp)


def _final(h, a, sp, batch, wl1, bl1, gl, bel, wl2, bl2):
  n, d = h.shape
  o = wl2.shape[1]

  def body(h_r, a_r, sp_r, b_r, wl1_r, bl1_r, gl_r, bel_r, wl2_r, bl2_r,
           out_r):
    s = jnp.sum(sp_r[...], axis=0)[:, None]
    hp = h_r[...] * (a_r[...] * s)
    gids = lax.broadcasted_iota(jnp.int32, (1, NUM_GRAPHS), 1)
    onehot = (b_r[...] == gids).astype(jnp.float32)
    gph = lax.dot_general(onehot, hp, (((0,), (0,)), ((), ())),
                          preferred_element_type=jnp.float32)
    t = jnp.dot(gph, wl1_r[...], preferred_element_type=jnp.float32) + bl1_r[...]
    t = _bn_relu(t, gl_r[...], bel_r[...], float(NUM_GRAPHS))
    out = jnp.dot(t, wl2_r[...], preferred_element_type=jnp.float32) + bl2_r[...]
    mx = jnp.max(out, axis=1, keepdims=True)
    z = out - mx
    out_r[...] = z - jnp.log(jnp.sum(jnp.exp(z), axis=1, keepdims=True))

  return pl.pallas_call(
      body, out_shape=jax.ShapeDtypeStruct((NUM_GRAPHS, o), jnp.float32))(
          h, a, sp, batch, wl1, bl1, gl, bel, wl2, bl2)


def kernel(x, edge_index, batch, W1a, b1a, g1, be1, W1b, b1b, Wa1, ba1,
           W2a, b2a, g2, be2, W2b, b2b, Wa2, ba2, Wl1, bl1, gl, bel, Wl2, bl2):
  n, d = x.shape
  e = edge_index.shape[1]
  row = edge_index[0].astype(jnp.int32)
  col = edge_index[1].astype(jnp.int32)
  row4 = row.reshape(NW, 10, 25, 40)
  col4 = col.reshape(NW, 10, 25, 40)
  row3 = row.reshape(NW, 125, 80)
  col3 = col.reshape(NW, 125, 80)
  batch2 = batch.astype(jnp.int32).reshape(n, 1)
  npad = ((n + 127) // 128) * 128
  zeros = jnp.zeros((npad, d), jnp.float32)
  zeros1 = jnp.zeros((npad,), jnp.float32)

  p1 = _seg_rows(x, row4, col4, zeros)[:, :n]
  h1, a1 = _dense_conv(x, p1, W1a, b1a, g1, be1, W1b, b1b, Wa1, ba1)
  s1p = _seg_scalar(a1.reshape(n), row3, col3, zeros1)
  h1p = _scale_rows(h1, a1, s1p)
  p2 = _seg_rows(h1p, row4, col4, zeros)[:, :n]
  h2, a2 = _dense_conv(h1p, p2, W2a, b2a, g2, be2, W2b, b2b, Wa2, ba2)
  s2p = _seg_scalar(a2.reshape(n), row3, col3, zeros1)
  return _final(h2, a2, s2p, batch2, Wl1, bl1, gl, bel, Wl2, bl2)


# speedup vs baseline: 1.2824x; 1.0265x over previous
import functools

import jax
import jax.numpy as jnp
from jax import lax
from jax.experimental import pallas as pl
from jax.experimental.pallas import tpu as pltpu
from jax.experimental.pallas import tpu_sc as plsc

NC = 2
NS = 16
NW = NC * NS
LANES = 16
CH = 128
NUM_GRAPHS = 128


def _seg_rows(table, row4, col4, zeros):
  n, d = table.shape
  npad = zeros.shape[0]
  _, nblocks, bpb, ch = row4.shape
  nb = 5
  rpb = bpb // nb
  nbodies = nblocks // 2
  rpt = npad // NS
  mesh = plsc.VectorSubcoreMesh(core_axis_name="c", subcore_axis_name="s")

  @functools.partial(
      pl.kernel, mesh=mesh,
      out_type=jax.ShapeDtypeStruct((NC, npad, d), jnp.float32),
      scratch_types=[
          pltpu.VMEM((bpb, ch), jnp.int32),
          pltpu.VMEM((bpb, ch), jnp.int32),
          pltpu.VMEM((bpb, ch), jnp.int32),
          pltpu.VMEM((bpb, ch), jnp.int32),
      ] + [pltpu.VMEM((ch, d), jnp.float32) for _ in range(nb)] + [
          pltpu.VMEM_SHARED((npad, d), jnp.float32),
      ] + [pltpu.SemaphoreType.DMA for _ in range(2 * nb + 2)])
  def k(table_h, row_h, col_h, zeros_h, out_h, rowA, colA, rowB, colB, *rest):
    bufs = rest[:nb]
    acc = rest[nb]
    gsems = rest[nb + 1:2 * nb + 1]
    ssems = rest[2 * nb + 1:3 * nb + 1]
    isemA, isemB = rest[3 * nb + 1:]
    c = lax.axis_index("c")
    s = lax.axis_index("s")
    wid = s * NC + c
    last = nblocks - 1
    pltpu.async_copy(row_h.at[wid, 0], rowA, isemA)
    pltpu.async_copy(col_h.at[wid, 0], colA, isemA)
    pltpu.async_copy(row_h.at[wid, 1], rowB, isemB)
    pltpu.async_copy(col_h.at[wid, 1], colB, isemB)
    pltpu.sync_copy(zeros_h.at[pl.ds(s * rpt, rpt)], acc.at[pl.ds(s * rpt, rpt)])
    plsc.subcore_barrier()

    def do_block(rowi, coli):
      for r in range(rpb):
        gs = [pltpu.async_copy(table_h.at[rowi.at[r * nb + b]], bufs[b],
                               gsems[b]) for b in range(nb)]
        ss = []
        for b in range(nb):
          gs[b].wait()
          ss.append(pltpu.async_copy(bufs[b], acc.at[coli.at[r * nb + b]],
                                     ssems[b], add=True))
        for b in range(nb):
          ss[b].wait()

    def drain_fetch(rowi, coli, isem, nxt):
      pltpu.make_async_copy(row_h.at[wid, 0], rowi, isem).wait()
      pltpu.make_async_copy(col_h.at[wid, 0], coli, isem).wait()
      return nxt

    def body(j, _):
      drain_fetch(rowA, colA, isemA, None)
      do_block(rowA, colA)
      nxtA = jnp.minimum(2 * j + 2, last)
      pltpu.async_copy(row_h.at[wid, nxtA], rowA, isemA)
      pltpu.async_copy(col_h.at[wid, nxtA], colA, isemA)
      drain_fetch(rowB, colB, isemB, None)
      do_block(rowB, colB)
      nxtB = jnp.minimum(2 * j + 3, last)
      pltpu.async_copy(row_h.at[wid, nxtB], rowB, isemB)
      pltpu.async_copy(col_h.at[wid, nxtB], colB, isemB)
      return ()

    lax.fori_loop(0, nbodies, body, ())
    drain_fetch(rowA, colA, isemA, None)
    drain_fetch(rowB, colB, isemB, None)
    plsc.subcore_barrier()
    pltpu.sync_copy(acc.at[pl.ds(s * rpt, rpt)],
                    out_h.at[c, pl.ds(s * rpt, rpt)])

  return k(table, row4, col4, zeros)


def _seg_scalar(attn, row3, col3, zeros1):
  n = attn.shape[0]
  npad = zeros1.shape[0]
  nchunks = row3.shape[1]
  ch = row3.shape[2]
  nb = 25
  rounds = nchunks // nb
  rpt = npad // NS
  mesh = plsc.VectorSubcoreMesh(core_axis_name="c", subcore_axis_name="s")

  @functools.partial(
      pl.kernel, mesh=mesh,
      out_type=jax.ShapeDtypeStruct((NC * npad,), jnp.float32),
      scratch_types=[
          pltpu.VMEM((nchunks, ch), jnp.int32),
          pltpu.VMEM((nchunks, ch), jnp.int32),
          pltpu.VMEM((rpt,), jnp.float32),
      ] + [pltpu.VMEM((ch,), jnp.float32) for _ in range(nb)] + [
          pltpu.VMEM_SHARED((npad,), jnp.float32),
          pltpu.SemaphoreType.DMA,
          pltpu.SemaphoreType.DMA,
      ])
  def k(attn_h, row_h, col_h, zeros_h, out_h, rowi, coli, zbuf, *rest):
    bufs = rest[:nb]
    acc = rest[nb]
    gsem = rest[nb + 1]
    ssem = rest[nb + 2]
    c = lax.axis_index("c")
    s = lax.axis_index("s")
    wid = s * NC + c
    pltpu.sync_copy(row_h.at[wid], rowi)
    pltpu.sync_copy(col_h.at[wid], coli)
    pltpu.sync_copy(zeros_h.at[pl.ds(s * rpt, rpt)], zbuf)
    pltpu.sync_copy(zbuf, acc.at[pl.ds(s * rpt, rpt)])
    plsc.subcore_barrier()

    def body(j, _):
      base = j * nb
      gs = [pltpu.async_copy(attn_h.at[rowi.at[base + b]], bufs[b], gsem)
            for b in range(nb)]
      for b in range(nb):
        gs[b].wait()
      ss = [pltpu.async_copy(bufs[b], acc.at[coli.at[base + b]], ssem,
                             add=True) for b in range(nb)]
      for b in range(nb):
        ss[b].wait()
      return ()

    lax.fori_loop(0, rounds, body, ())
    plsc.subcore_barrier()
    pltpu.sync_copy(acc.at[pl.ds(s * rpt, rpt)], zbuf)
    pltpu.sync_copy(zbuf, out_h.at[pl.ds(c * npad + s * rpt, rpt)])

  return k(attn, row3, col3, zeros1).reshape(NC, npad)[:, :n]


def _bn_relu(t, g, b, n_rows):
  m = jnp.sum(t, axis=0, keepdims=True) * (1.0 / n_rows)
  v = jnp.sum((t - m) ** 2, axis=0, keepdims=True) * (1.0 / n_rows)
  return jnp.maximum((t - m) * lax.rsqrt(v + 1e-5) * g + b, 0.0)


def _dense_conv(x, p, wa, ba, g, be, wb, bb, watt, batt):
  n, d = x.shape
  h_dim = wb.shape[1]

  def body(x_r, p_r, wa_r, ba_r, g_r, be_r, wb_r, bb_r, watt_r, batt_r,
           h_out, a_out):
    xin = x_r[...] + p_r[0] + p_r[1]
    t = jnp.dot(xin, wa_r[...], preferred_element_type=jnp.float32) + ba_r[...]
    t = _bn_relu(t, g_r[...], be_r[...], float(n))
    h = jnp.dot(t, wb_r[...], preferred_element_type=jnp.float32) + bb_r[...]
    hr = jnp.maximum(h, 0.0)
    h_out[...] = hr
    logit = jnp.dot(hr, watt_r[...], preferred_element_type=jnp.float32)
    a_out[...] = jax.nn.sigmoid(logit + batt_r[...])

  return pl.pallas_call(
      body,
      out_shape=(jax.ShapeDtypeStruct((n, h_dim), jnp.float32),
                 jax.ShapeDtypeStruct((n, 1), jnp.float32)),
  )(x, p, wa, ba, g, be, wb, bb, watt, batt)


def _scale_rows(h, a, sp):
  n, d = h.shape

  def body(h_r, a_r, sp_r, out_r):
    s = jnp.sum(sp_r[...], axis=0)[:, None]
    out_r[...] = h_r[...] * (a_r[...] * s)

  return pl.pallas_call(
      body, out_shape=jax.ShapeDtypeStruct((n, d), jnp.float32))(h, a, sp)


def _final(h, a, sp, batch, wl1, bl1, gl, bel, wl2, bl2):
  n, d = h.shape
  o = wl2.shape[1]

  def body(h_r, a_r, sp_r, b_r, wl1_r, bl1_r, gl_r, bel_r, wl2_r, bl2_r,
           out_r):
    s = jnp.sum(sp_r[...], axis=0)[:, None]
    hp = h_r[...] * (a_r[...] * s)
    gids = lax.broadcasted_iota(jnp.int32, (1, NUM_GRAPHS), 1)
    onehot = (b_r[...] == gids).astype(jnp.float32)
    gph = lax.dot_general(onehot, hp, (((0,), (0,)), ((), ())),
                          preferred_element_type=jnp.float32)
    t = jnp.dot(gph, wl1_r[...], preferred_element_type=jnp.float32) + bl1_r[...]
    t = _bn_relu(t, gl_r[...], bel_r[...], float(NUM_GRAPHS))
    out = jnp.dot(t, wl2_r[...], preferred_element_type=jnp.float32) + bl2_r[...]
    mx = jnp.max(out, axis=1, keepdims=True)
    z = out - mx
    out_r[...] = z - jnp.log(jnp.sum(jnp.exp(z), axis=1, keepdims=True))

  return pl.pallas_call(
      body, out_shape=jax.ShapeDtypeStruct((NUM_GRAPHS, o), jnp.float32))(
          h, a, sp, batch, wl1, bl1, gl, bel, wl2, bl2)


def kernel(x, edge_index, batch, W1a, b1a, g1, be1, W1b, b1b, Wa1, ba1,
           W2a, b2a, g2, be2, W2b, b2b, Wa2, ba2, Wl1, bl1, gl, bel, Wl2, bl2):
  n, d = x.shape
  e = edge_index.shape[1]
  row = edge_index[0].astype(jnp.int32)
  col = edge_index[1].astype(jnp.int32)
  row4 = row.reshape(NW, 10, 20, 50)
  col4 = col.reshape(NW, 10, 20, 50)
  row3 = row.reshape(NW, 125, 80)
  col3 = col.reshape(NW, 125, 80)
  batch2 = batch.astype(jnp.int32).reshape(n, 1)
  npad = ((n + 127) // 128) * 128
  zeros = jnp.zeros((npad, d), jnp.float32)
  zeros1 = jnp.zeros((npad,), jnp.float32)

  p1 = _seg_rows(x, row4, col4, zeros)[:, :n]
  h1, a1 = _dense_conv(x, p1, W1a, b1a, g1, be1, W1b, b1b, Wa1, ba1)
  s1p = _seg_scalar(a1.reshape(n), row3, col3, zeros1)
  h1p = _scale_rows(h1, a1, s1p)
  p2 = _seg_rows(h1p, row4, col4, zeros)[:, :n]
  h2, a2 = _dense_conv(h1p, p2, W2a, b2a, g2, be2, W2b, b2b, Wa2, ba2)
  s2p = _seg_scalar(a2.reshape(n), row3, col3, zeros1)
  return _final(h2, a2, s2p, batch2, Wl1, bl1, gl, bel, Wl2, bl2)


# speedup vs baseline: 1.3338x; 1.0401x over previous
import functools

import jax
import jax.numpy as jnp
from jax import lax
from jax.experimental import pallas as pl
from jax.experimental.pallas import tpu as pltpu
from jax.experimental.pallas import tpu_sc as plsc

NC = 2
NS = 16
NW = NC * NS
LANES = 16
CH = 128
NUM_GRAPHS = 128


def _seg_rows(table, row4, col4, zeros):
  n, d = table.shape
  npad = zeros.shape[0]
  _, nblocks, bpb, ch = row4.shape
  nb = 5
  rpb = bpb // nb
  nbodies = nblocks // 2
  rpt = npad // NS
  mesh = plsc.VectorSubcoreMesh(core_axis_name="c", subcore_axis_name="s")

  @functools.partial(
      pl.kernel, mesh=mesh,
      out_type=jax.ShapeDtypeStruct((NC, npad, d), jnp.float32),
      scratch_types=[
          pltpu.VMEM((bpb, ch), jnp.int32),
          pltpu.VMEM((bpb, ch), jnp.int32),
          pltpu.VMEM((bpb, ch), jnp.int32),
          pltpu.VMEM((bpb, ch), jnp.int32),
      ] + [pltpu.VMEM((ch, d), jnp.float32) for _ in range(nb)] + [
          pltpu.VMEM_SHARED((npad, d), jnp.float32),
      ] + [pltpu.SemaphoreType.DMA for _ in range(2 * nb + 2)])
  def k(table_h, row_h, col_h, zeros_h, out_h, rowA, colA, rowB, colB, *rest):
    bufs = rest[:nb]
    acc = rest[nb]
    gsems = rest[nb + 1:2 * nb + 1]
    ssems = rest[2 * nb + 1:3 * nb + 1]
    isemA, isemB = rest[3 * nb + 1:]
    c = lax.axis_index("c")
    s = lax.axis_index("s")
    wid = s * NC + c
    last = nblocks - 1
    pltpu.async_copy(row_h.at[wid, 0], rowA, isemA)
    pltpu.async_copy(col_h.at[wid, 0], colA, isemA)
    pltpu.async_copy(row_h.at[wid, 1], rowB, isemB)
    pltpu.async_copy(col_h.at[wid, 1], colB, isemB)
    pltpu.sync_copy(zeros_h.at[pl.ds(s * rpt, rpt)], acc.at[pl.ds(s * rpt, rpt)])
    plsc.subcore_barrier()

    def do_block(rowi, coli):
      for r in range(rpb):
        gs = [pltpu.async_copy(table_h.at[rowi.at[r * nb + b]], bufs[b],
                               gsems[b]) for b in range(nb)]
        ss = []
        for b in range(nb):
          gs[b].wait()
          ss.append(pltpu.async_copy(bufs[b], acc.at[coli.at[r * nb + b]],
                                     ssems[b], add=True))
        for b in range(nb):
          ss[b].wait()

    def drain_fetch(rowi, coli, isem, nxt):
      pltpu.make_async_copy(row_h.at[wid, 0], rowi, isem).wait()
      pltpu.make_async_copy(col_h.at[wid, 0], coli, isem).wait()
      return nxt

    def body(j, _):
      drain_fetch(rowA, colA, isemA, None)
      do_block(rowA, colA)
      nxtA = jnp.minimum(2 * j + 2, last)
      pltpu.async_copy(row_h.at[wid, nxtA], rowA, isemA)
      pltpu.async_copy(col_h.at[wid, nxtA], colA, isemA)
      drain_fetch(rowB, colB, isemB, None)
      do_block(rowB, colB)
      nxtB = jnp.minimum(2 * j + 3, last)
      pltpu.async_copy(row_h.at[wid, nxtB], rowB, isemB)
      pltpu.async_copy(col_h.at[wid, nxtB], colB, isemB)
      return ()

    lax.fori_loop(0, nbodies, body, ())
    drain_fetch(rowA, colA, isemA, None)
    drain_fetch(rowB, colB, isemB, None)
    plsc.subcore_barrier()
    pltpu.sync_copy(acc.at[pl.ds(s * rpt, rpt)],
                    out_h.at[c, pl.ds(s * rpt, rpt)])

  return k(table, row4, col4, zeros)


def _seg_scalar(attn, row3, col3, zeros1):
  n = attn.shape[0]
  npad = zeros1.shape[0]
  nchunks = row3.shape[1]
  ch = row3.shape[2]
  nb = 25
  rounds = nchunks // nb
  rpt = npad // NS
  mesh = plsc.VectorSubcoreMesh(core_axis_name="c", subcore_axis_name="s")

  @functools.partial(
      pl.kernel, mesh=mesh,
      out_type=jax.ShapeDtypeStruct((NC * npad,), jnp.float32),
      scratch_types=[
          pltpu.VMEM((nchunks, ch), jnp.int32),
          pltpu.VMEM((nchunks, ch), jnp.int32),
          pltpu.VMEM((rpt,), jnp.float32),
      ] + [pltpu.VMEM((ch,), jnp.float32) for _ in range(nb)] + [
          pltpu.VMEM_SHARED((npad,), jnp.float32),
          pltpu.SemaphoreType.DMA,
          pltpu.SemaphoreType.DMA,
      ])
  def k(attn_h, row_h, col_h, zeros_h, out_h, rowi, coli, zbuf, *rest):
    bufs = rest[:nb]
    acc = rest[nb]
    gsem = rest[nb + 1]
    ssem = rest[nb + 2]
    c = lax.axis_index("c")
    s = lax.axis_index("s")
    wid = s * NC + c
    pltpu.sync_copy(row_h.at[wid], rowi)
    pltpu.sync_copy(col_h.at[wid], coli)
    pltpu.sync_copy(zeros_h.at[pl.ds(s * rpt, rpt)], zbuf)
    pltpu.sync_copy(zbuf, acc.at[pl.ds(s * rpt, rpt)])
    plsc.subcore_barrier()

    def body(j, _):
      base = j * nb
      gs = [pltpu.async_copy(attn_h.at[rowi.at[base + b]], bufs[b], gsem)
            for b in range(nb)]
      for b in range(nb):
        gs[b].wait()
      ss = [pltpu.async_copy(bufs[b], acc.at[coli.at[base + b]], ssem,
                             add=True) for b in range(nb)]
      for b in range(nb):
        ss[b].wait()
      return ()

    lax.fori_loop(0, rounds, body, ())
    plsc.subcore_barrier()
    pltpu.sync_copy(acc.at[pl.ds(s * rpt, rpt)], zbuf)
    pltpu.sync_copy(zbuf, out_h.at[pl.ds(c * npad + s * rpt, rpt)])

  return k(attn, row3, col3, zeros1).reshape(NC, npad)


def _bn_relu(t, g, b, n_rows):
  m = jnp.sum(t, axis=0, keepdims=True) * (1.0 / n_rows)
  v = jnp.sum((t - m) ** 2, axis=0, keepdims=True) * (1.0 / n_rows)
  return jnp.maximum((t - m) * lax.rsqrt(v + 1e-5) * g + b, 0.0)


def _dense_conv(x, p, wa, ba, g, be, wb, bb, watt, batt):
  n, d = x.shape
  h_dim = wb.shape[1]

  def body(x_r, p_r, wa_r, ba_r, g_r, be_r, wb_r, bb_r, watt_r, batt_r,
           h_out, a_out):
    xin = x_r[...] + p_r[0, :n, :] + p_r[1, :n, :]
    t = jnp.dot(xin, wa_r[...], preferred_element_type=jnp.float32) + ba_r[...]
    t = _bn_relu(t, g_r[...], be_r[...], float(n))
    h = jnp.dot(t, wb_r[...], preferred_element_type=jnp.float32) + bb_r[...]
    hr = jnp.maximum(h, 0.0)
    h_out[...] = hr
    logit = jnp.dot(hr, watt_r[...], preferred_element_type=jnp.float32)
    a_out[...] = jax.nn.sigmoid(logit + batt_r[...])

  return pl.pallas_call(
      body,
      out_shape=(jax.ShapeDtypeStruct((n, h_dim), jnp.float32),
                 jax.ShapeDtypeStruct((n, 1), jnp.float32)),
  )(x, p, wa, ba, g, be, wb, bb, watt, batt)


def _scale_rows(h, a, sp):
  n, d = h.shape

  def body(h_r, a_r, sp_r, out_r):
    s = (sp_r[0, :n] + sp_r[1, :n])[:, None]
    out_r[...] = h_r[...] * (a_r[...] * s)

  return pl.pallas_call(
      body, out_shape=jax.ShapeDtypeStruct((n, d), jnp.float32))(h, a, sp)


def _final(h, a, sp, batch, wl1, bl1, gl, bel, wl2, bl2):
  n, d = h.shape
  o = wl2.shape[1]

  def body(h_r, a_r, sp_r, b_r, wl1_r, bl1_r, gl_r, bel_r, wl2_r, bl2_r,
           out_r):
    s = (sp_r[0, :n] + sp_r[1, :n])[:, None]
    hp = h_r[...] * (a_r[...] * s)
    gids = lax.broadcasted_iota(jnp.int32, (1, NUM_GRAPHS), 1)
    onehot = (b_r[...] == gids).astype(jnp.float32)
    gph = lax.dot_general(onehot, hp, (((0,), (0,)), ((), ())),
                          preferred_element_type=jnp.float32)
    t = jnp.dot(gph, wl1_r[...], preferred_element_type=jnp.float32) + bl1_r[...]
    t = _bn_relu(t, gl_r[...], bel_r[...], float(NUM_GRAPHS))
    out = jnp.dot(t, wl2_r[...], preferred_element_type=jnp.float32) + bl2_r[...]
    mx = jnp.max(out, axis=1, keepdims=True)
    z = out - mx
    out_r[...] = z - jnp.log(jnp.sum(jnp.exp(z), axis=1, keepdims=True))

  return pl.pallas_call(
      body, out_shape=jax.ShapeDtypeStruct((NUM_GRAPHS, o), jnp.float32))(
          h, a, sp, batch, wl1, bl1, gl, bel, wl2, bl2)


def kernel(x, edge_index, batch, W1a, b1a, g1, be1, W1b, b1b, Wa1, ba1,
           W2a, b2a, g2, be2, W2b, b2b, Wa2, ba2, Wl1, bl1, gl, bel, Wl2, bl2):
  n, d = x.shape
  e = edge_index.shape[1]
  row = edge_index[0].astype(jnp.int32)
  col = edge_index[1].astype(jnp.int32)
  row4 = row.reshape(NW, 10, 20, 50)
  col4 = col.reshape(NW, 10, 20, 50)
  row3 = row.reshape(NW, 125, 80)
  col3 = col.reshape(NW, 125, 80)
  batch2 = batch.astype(jnp.int32).reshape(n, 1)
  npad = ((n + 127) // 128) * 128
  zeros = jnp.zeros((npad, d), jnp.float32)
  zeros1 = jnp.zeros((npad,), jnp.float32)

  p1 = _seg_rows(x, row4, col4, zeros)
  h1, a1 = _dense_conv(x, p1, W1a, b1a, g1, be1, W1b, b1b, Wa1, ba1)
  s1p = _seg_scalar(a1.reshape(n), row3, col3, zeros1)
  h1p = _scale_rows(h1, a1, s1p)
  p2 = _seg_rows(h1p, row4, col4, zeros)
  h2, a2 = _dense_conv(h1p, p2, W2a, b2a, g2, be2, W2b, b2b, Wa2, ba2)
  s2p = _seg_scalar(a2.reshape(n), row3, col3, zeros1)
  return _final(h2, a2, s2p, batch2, Wl1, bl1, gl, bel, Wl2, bl2)
